# baseline (speedup 1.0000x reference)
import functools
import math

import jax
import jax.numpy as jnp
from jax.experimental import pallas as pl
from jax.experimental.pallas import tpu as pltpu

DIM = 768
HIDDEN = 1536
E = 8
MAX_DEPTH = 32
ROPE_BASE = 10000.0
T = 2048
EPAD = 128

_RT_BLK = 256
_XB_BLK = 1024


def _router_body(x_ref, wr_ref, ek_ref, bias_ref, trig_ref, gates_ref):
    x = x_ref[...]
    q = jax.lax.dot_general(x, wr_ref[...], (((1,), (1,)), ((), ())),
                            preferred_element_type=jnp.float32)
    eps = jnp.finfo(jnp.float32).eps
    q = q * jax.lax.rsqrt(jnp.mean(q * q, axis=1, keepdims=True) + eps)
    half = DIM // 2
    q1 = q[:, :half]
    q2 = q[:, half:]
    cf = trig_ref[0:1, :]
    sf = trig_ref[1:2, :]
    cr = trig_ref[2:3, :]
    sr = trig_ref[3:4, :]
    qr = jnp.concatenate([q1 * cf + q2 * sf, -q1 * sr + q2 * cr], axis=1)
    logits = jax.lax.dot_general(qr, ek_ref[...], (((1,), (1,)), ((), ())),
                                 preferred_element_type=jnp.float32)
    logits = logits * (1.0 / math.sqrt(DIM))
    g = jax.nn.sigmoid(logits)
    biased = logits + bias_ref[0:1, :]
    lane = jax.lax.broadcasted_iota(jnp.int32, biased.shape, 1)
    m1 = jnp.max(biased, axis=1, keepdims=True)
    a1 = jnp.min(jnp.where(biased == m1, lane, EPAD), axis=1, keepdims=True)
    masked = jnp.where(lane == a1, -jnp.inf, biased)
    m2 = jnp.max(masked, axis=1, keepdims=True)
    a2 = jnp.min(jnp.where(masked == m2, lane, EPAD), axis=1, keepdims=True)
    g1 = jnp.sum(jnp.where(lane == a1, g, 0.0), axis=1, keepdims=True)
    g2 = jnp.sum(jnp.where(lane == a2, g, 0.0), axis=1, keepdims=True)
    den = jnp.maximum(g1 + g2, 1e-9)
    gates_ref[...] = jnp.where(
        lane == a1, g1 / den, jnp.where(lane == a2, g2 / den, 0.0))


def _expert_body(x_ref, gates_ref, wfc_ref, wproj_ref, out_ref):
    e = pl.program_id(1)
    xb = x_ref[...].astype(jnp.bfloat16)
    h = jax.lax.dot_general(xb, wfc_ref[0], (((1,), (1,)), ((), ())),
                            preferred_element_type=jnp.float32)
    h = jnp.square(jnp.maximum(h, 0.0)).astype(jnp.bfloat16)
    o = jax.lax.dot_general(h, wproj_ref[0], (((1,), (1,)), ((), ())),
                            preferred_element_type=jnp.float32)
    lane = jax.lax.broadcasted_iota(jnp.int32, gates_ref.shape, 1)
    ge = jnp.sum(jnp.where(lane == e, gates_ref[...], 0.0), axis=1,
                 keepdims=True)

    @pl.when(e == 0)
    def _():
        out_ref[...] = ge * o

    @pl.when(e > 0)
    def _():
        out_ref[...] += ge * o


def kernel(x, depth_idx, W_router, expert_keys, expert_bias, W_fc, W_proj):
    bsz, seqlen, dim = x.shape
    flat_x = x.reshape(bsz * seqlen, dim)
    t = flat_x.shape[0]

    inv_freq = 1.0 / (ROPE_BASE ** (jnp.arange(0, dim, 2, dtype=jnp.float32) / dim))
    d = jnp.asarray(depth_idx, jnp.float32)
    fwd = d * inv_freq
    rev = (jnp.float32(MAX_DEPTH - 1) - d) * inv_freq
    trig = jnp.zeros((8, dim // 2), jnp.float32)
    trig = trig.at[0].set(jnp.cos(fwd)).at[1].set(jnp.sin(fwd))
    trig = trig.at[2].set(jnp.cos(rev)).at[3].set(jnp.sin(rev))
    ek_pad = jnp.zeros((EPAD, dim), jnp.float32).at[:E].set(expert_keys)
    bias_pad = jnp.full((EPAD,), -1e30, jnp.float32).at[:E].set(expert_bias)
    bias_pad = jnp.broadcast_to(bias_pad[None, :], (8, EPAD))

    n_rt = t // _RT_BLK
    gates = pl.pallas_call(
        _router_body,
        grid=(n_rt,),
        in_specs=[
            pl.BlockSpec((_RT_BLK, dim), lambda i: (i, 0)),
            pl.BlockSpec((dim, dim), lambda i: (0, 0)),
            pl.BlockSpec((EPAD, dim), lambda i: (0, 0)),
            pl.BlockSpec((8, EPAD), lambda i: (0, 0)),
            pl.BlockSpec((8, dim // 2), lambda i: (0, 0)),
        ],
        out_specs=pl.BlockSpec((_RT_BLK, EPAD), lambda i: (i, 0)),
        out_shape=jax.ShapeDtypeStruct((t, EPAD), jnp.float32),
    )(flat_x, W_router, ek_pad, bias_pad, trig)

    n_xb = t // _XB_BLK
    y = pl.pallas_call(
        _expert_body,
        grid=(n_xb, E),
        in_specs=[
            pl.BlockSpec((_XB_BLK, dim), lambda i, e: (i, 0)),
            pl.BlockSpec((_XB_BLK, EPAD), lambda i, e: (i, 0)),
            pl.BlockSpec((1, HIDDEN, dim), lambda i, e: (e, 0, 0)),
            pl.BlockSpec((1, dim, HIDDEN), lambda i, e: (e, 0, 0)),
        ],
        out_specs=pl.BlockSpec((_XB_BLK, dim), lambda i, e: (i, 0)),
        out_shape=jax.ShapeDtypeStruct((t, dim), jnp.float32),
    )(flat_x, gates, W_fc.astype(jnp.bfloat16), W_proj.astype(jnp.bfloat16))

    return y.reshape(bsz, seqlen, dim)


# speedup vs baseline: 1.3205x; 1.3205x over previous
import functools
import math

import jax
import jax.numpy as jnp
from jax.experimental import pallas as pl
from jax.experimental.pallas import tpu as pltpu

DIM = 768
HIDDEN = 1536
E = 8
MAX_DEPTH = 32
ROPE_BASE = 10000.0
T = 2048
EPAD = 128

_RT_BLK = 256
_XB_BLK = 2048


def _router_body(x_ref, wr_ref, ek_ref, bias_ref, trig_ref, gates_ref):
    x = x_ref[...]
    q = jax.lax.dot_general(x, wr_ref[...], (((1,), (1,)), ((), ())),
                            preferred_element_type=jnp.float32)
    eps = jnp.finfo(jnp.float32).eps
    q = q * jax.lax.rsqrt(jnp.mean(q * q, axis=1, keepdims=True) + eps)
    half = DIM // 2
    q1 = q[:, :half]
    q2 = q[:, half:]
    cf = trig_ref[0:1, :]
    sf = trig_ref[1:2, :]
    cr = trig_ref[2:3, :]
    sr = trig_ref[3:4, :]
    qr = jnp.concatenate([q1 * cf + q2 * sf, -q1 * sr + q2 * cr], axis=1)
    logits = jax.lax.dot_general(qr, ek_ref[...], (((1,), (1,)), ((), ())),
                                 preferred_element_type=jnp.float32)
    logits = logits * (1.0 / math.sqrt(DIM))
    g = jax.nn.sigmoid(logits)
    biased = logits + bias_ref[0:1, :]
    lane = jax.lax.broadcasted_iota(jnp.int32, biased.shape, 1)
    m1 = jnp.max(biased, axis=1, keepdims=True)
    a1 = jnp.min(jnp.where(biased == m1, lane, EPAD), axis=1, keepdims=True)
    masked = jnp.where(lane == a1, -jnp.inf, biased)
    m2 = jnp.max(masked, axis=1, keepdims=True)
    a2 = jnp.min(jnp.where(masked == m2, lane, EPAD), axis=1, keepdims=True)
    g1 = jnp.sum(jnp.where(lane == a1, g, 0.0), axis=1, keepdims=True)
    g2 = jnp.sum(jnp.where(lane == a2, g, 0.0), axis=1, keepdims=True)
    den = jnp.maximum(g1 + g2, 1e-9)
    gates_ref[...] = jnp.where(
        lane == a1, g1 / den, jnp.where(lane == a2, g2 / den, 0.0))


def _expert_body(x_ref, gates_ref, wfc_ref, wproj_ref, out_ref):
    e = pl.program_id(1)
    h = jax.lax.dot_general(x_ref[...], wfc_ref[0], (((1,), (1,)), ((), ())),
                            preferred_element_type=jnp.float32)
    h = jnp.square(jnp.maximum(h, 0.0))
    o = jax.lax.dot_general(h, wproj_ref[0], (((1,), (1,)), ((), ())),
                            preferred_element_type=jnp.float32)
    lane = jax.lax.broadcasted_iota(jnp.int32, gates_ref.shape, 1)
    ge = jnp.sum(jnp.where(lane == e, gates_ref[...], 0.0), axis=1,
                 keepdims=True)

    @pl.when(e == 0)
    def _():
        out_ref[...] = ge * o

    @pl.when(e > 0)
    def _():
        out_ref[...] += ge * o


def kernel(x, depth_idx, W_router, expert_keys, expert_bias, W_fc, W_proj):
    bsz, seqlen, dim = x.shape
    flat_x = x.reshape(bsz * seqlen, dim)
    t = flat_x.shape[0]

    inv_freq = 1.0 / (ROPE_BASE ** (jnp.arange(0, dim, 2, dtype=jnp.float32) / dim))
    d = jnp.asarray(depth_idx, jnp.float32)
    fwd = d * inv_freq
    rev = (jnp.float32(MAX_DEPTH - 1) - d) * inv_freq
    trig = jnp.zeros((8, dim // 2), jnp.float32)
    trig = trig.at[0].set(jnp.cos(fwd)).at[1].set(jnp.sin(fwd))
    trig = trig.at[2].set(jnp.cos(rev)).at[3].set(jnp.sin(rev))
    ek_pad = jnp.zeros((EPAD, dim), jnp.float32).at[:E].set(expert_keys)
    bias_pad = jnp.full((EPAD,), -1e30, jnp.float32).at[:E].set(expert_bias)
    bias_pad = jnp.broadcast_to(bias_pad[None, :], (8, EPAD))

    n_rt = t // _RT_BLK
    gates = pl.pallas_call(
        _router_body,
        grid=(n_rt,),
        in_specs=[
            pl.BlockSpec((_RT_BLK, dim), lambda i: (i, 0)),
            pl.BlockSpec((dim, dim), lambda i: (0, 0)),
            pl.BlockSpec((EPAD, dim), lambda i: (0, 0)),
            pl.BlockSpec((8, EPAD), lambda i: (0, 0)),
            pl.BlockSpec((8, dim // 2), lambda i: (0, 0)),
        ],
        out_specs=pl.BlockSpec((_RT_BLK, EPAD), lambda i: (i, 0)),
        out_shape=jax.ShapeDtypeStruct((t, EPAD), jnp.float32),
    )(flat_x, W_router, ek_pad, bias_pad, trig)

    n_xb = t // _XB_BLK
    y = pl.pallas_call(
        _expert_body,
        grid=(n_xb, E),
        in_specs=[
            pl.BlockSpec((_XB_BLK, dim), lambda i, e: (i, 0)),
            pl.BlockSpec((_XB_BLK, EPAD), lambda i, e: (i, 0)),
            pl.BlockSpec((1, HIDDEN, dim), lambda i, e: (e, 0, 0)),
            pl.BlockSpec((1, dim, HIDDEN), lambda i, e: (e, 0, 0)),
        ],
        out_specs=pl.BlockSpec((_XB_BLK, dim), lambda i, e: (i, 0)),
        out_shape=jax.ShapeDtypeStruct((t, dim), jnp.float32),
    )(flat_x, gates, W_fc, W_proj)

    return y.reshape(bsz, seqlen, dim)
